# baseline (speedup 1.0000x reference)
import functools

import jax
import jax.numpy as jnp
from jax.experimental import pallas as pl
from jax.experimental.pallas import tpu as pltpu

N = 2048
B = 8
LATENT = 4
NGMM = 10
NCHUNK = 16
CHUNK = N // NCHUNK


def _fused_body(g_ref, starts_ref,
                W1_ref, b1_ref, W2_ref, b2_ref, W3_ref, b3_ref,
                We1_ref, be1_ref, We2_ref, be2_ref,
                x_hbm, adj_hbm,
                out_ref, gamma_ref,
                x_vmem, adj_vmem, h1_vmem, xsem, sems):
    f32 = jnp.float32

    pltpu.make_async_copy(x_hbm, x_vmem, xsem).start()
    for c in range(NCHUNK):
        pltpu.make_async_copy(
            adj_hbm.at[pl.ds(c * CHUNK, CHUNK), :],
            adj_vmem.at[pl.ds(c * CHUNK, CHUNK), :],
            sems.at[c],
        ).start()

    pltpu.make_async_copy(x_hbm, x_vmem, xsem).wait()
    p1 = jnp.dot(x_vmem[...], W1_ref[...], preferred_element_type=f32)
    b1 = b1_ref[...]

    g = g_ref[...]
    starts = starts_ref[...]

    madj = jnp.zeros((B, N), dtype=f32)
    for c in range(NCHUNK):
        pltpu.make_async_copy(
            adj_hbm.at[pl.ds(c * CHUNK, CHUNK), :],
            adj_vmem.at[pl.ds(c * CHUNK, CHUNK), :],
            sems.at[c],
        ).wait()
        blk = adj_vmem[pl.ds(c * CHUNK, CHUNK), :]
        h1_vmem[pl.ds(c * CHUNK, CHUNK), :] = jnp.maximum(
            jnp.dot(blk, p1, preferred_element_type=f32) + b1, 0.0)
        pos = jax.lax.broadcasted_iota(jnp.int32, (B, CHUNK), 1) + c * CHUNK
        mask_c = ((pos >= starts) & (pos < g)).astype(f32)
        madj = madj + jnp.dot(mask_c, blk, preferred_element_type=f32)

    p2 = jnp.dot(h1_vmem[...], W2_ref[...], preferred_element_type=f32)
    b2 = b2_ref[...]
    W3 = W3_ref[...]

    pooled_acc = jnp.zeros((B, LATENT), dtype=f32)
    for c in range(NCHUNK):
        blk = adj_vmem[pl.ds(c * CHUNK, CHUNK), :]
        h2_blk = jnp.maximum(jnp.dot(blk, p2, preferred_element_type=f32) + b2, 0.0)
        p3_blk = jnp.dot(h2_blk, W3, preferred_element_type=f32)
        pooled_acc = pooled_acc + jnp.dot(
            madj[:, c * CHUNK:(c + 1) * CHUNK], p3_blk,
            preferred_element_type=f32)

    counts = (g - starts).astype(f32)
    pooled = pooled_acc / counts + b3_ref[...]

    hidden = jnp.maximum(jnp.dot(pooled, We1_ref[...], preferred_element_type=f32) + be1_ref[...], 0.0)
    logits = jnp.dot(hidden, We2_ref[...], preferred_element_type=f32) + be2_ref[...]
    m = jnp.max(logits, axis=1, keepdims=True)
    e = jnp.exp(logits - m)
    gamma = e / jnp.sum(e, axis=1, keepdims=True)

    out_ref[...] = pooled
    gamma_ref[...] = gamma


@functools.partial(jax.jit, static_argnames=("interpret",))
def _run(x, adj, g2, starts2, W1, b1, W2, b2, W3, b3, We1, be1, We2, be2,
         interpret=False):
    in_specs = (
        [pl.BlockSpec(memory_space=pltpu.MemorySpace.VMEM)] * 12
        + [pl.BlockSpec(memory_space=pl.ANY),
           pl.BlockSpec(memory_space=pl.ANY)])
    out, gamma = pl.pallas_call(
        _fused_body,
        out_shape=(
            jax.ShapeDtypeStruct((B, LATENT), jnp.float32),
            jax.ShapeDtypeStruct((B, NGMM), jnp.float32),
        ),
        in_specs=in_specs,
        scratch_shapes=[
            pltpu.VMEM((N, 512), jnp.float32),
            pltpu.VMEM((N, N), jnp.float32),
            pltpu.VMEM((N, 128), jnp.float32),
            pltpu.SemaphoreType.DMA,
            pltpu.SemaphoreType.DMA((NCHUNK,)),
        ],
        compiler_params=pltpu.CompilerParams(
            vmem_limit_bytes=100 * 1024 * 1024,
        ),
        interpret=interpret,
    )(g2, starts2,
      W1, b1.reshape(1, -1), W2, b2.reshape(1, -1), W3, b3.reshape(1, -1),
      We1, be1.reshape(1, -1), We2, be2.reshape(1, -1),
      x, adj)
    return out, gamma


def kernel(x, adj, graph_to_last_batch, W1, b1, W2, b2, W3, b3,
           We1, be1, We2, be2):
    g = graph_to_last_batch.astype(jnp.int32)
    starts = jnp.concatenate([jnp.zeros((1,), jnp.int32), g[:-1]])
    out, gamma = _run(x, adj, g.reshape(B, 1), starts.reshape(B, 1),
                      W1, b1, W2, b2, W3, b3, We1, be1, We2, be2)
    return (x, out, gamma)
